# baseline (speedup 1.0000x reference)
import functools

import jax
import jax.numpy as jnp
from jax import lax
from jax.experimental import pallas as pl
from jax.experimental.pallas import tpu as pltpu
from jax.experimental.pallas import tpu_sc as plsc

B = 16384
NUM_CONT = 16
NUM_CAT = 26
VOCAB = 100000
HIDDEN = 128
EMB_DIM = HIDDEN // 4
CONT_DIM = HIDDEN // 2
TOTAL = CONT_DIM + EMB_DIM * NUM_CAT

NC = 2
NS = 16
NW = NC * NS
ROWS = B * NUM_CAT
R_PER_W = ROWS // NW
CHUNK = 1024
N_CHUNKS = R_PER_W // CHUNK
IDX_ROWS = CHUNK // 128


def _gather_body(table_hbm, idx_hbm, out_hbm, idx_v, rows_v, sem):
    wid = lax.axis_index("s") * NC + lax.axis_index("c")
    base = wid * R_PER_W
    ibase = wid * (R_PER_W // 128)

    def body(c, carry):
        row0 = base + c * CHUNK
        irow0 = ibase + c * IDX_ROWS
        pltpu.sync_copy(idx_hbm.at[pl.ds(irow0, IDX_ROWS)], idx_v)
        copies = [
            pltpu.async_copy(
                table_hbm.at[idx_v.at[j]],
                rows_v.at[pl.ds(j * 128, 128)],
                sem,
            )
            for j in range(IDX_ROWS)
        ]
        for cp in copies:
            cp.wait()
        pltpu.sync_copy(rows_v, out_hbm.at[pl.ds(row0, CHUNK)])
        return carry

    lax.fori_loop(0, N_CHUNKS, body, 0)


_gather = functools.partial(
    pl.kernel,
    out_type=jax.ShapeDtypeStruct((ROWS, EMB_DIM), jnp.bfloat16),
    mesh=plsc.VectorSubcoreMesh(core_axis_name="c", subcore_axis_name="s"),
    scratch_types=[
        pltpu.VMEM((IDX_ROWS, 128), jnp.int32),
        pltpu.VMEM((CHUNK, EMB_DIM), jnp.bfloat16),
        pltpu.SemaphoreType.DMA,
    ],
    compiler_params=pltpu.CompilerParams(use_tc_tiling_on_sc=False),
)(_gather_body)


QA = 25088
NS_TR = 7
SLAB = QA // NS_TR


def _trans_body(in_ref, out_ref):
    s = pl.program_id(1)
    for k in range(NS_TR):
        @pl.when(s == k)
        def _(k=k):
            parts = []
            for a in range(4):
                lo = a * QA + k * SLAB
                hi = lo + SLAB
                if hi <= VOCAB:
                    parts.append(in_ref[0, :, lo:hi])
                else:
                    tail = in_ref[0, :, lo:VOCAB]
                    parts.append(jnp.concatenate(
                        [tail, jnp.zeros((EMB_DIM, hi - VOCAB),
                                         jnp.float32)], axis=1))
            x4 = jnp.concatenate(parts, axis=0)
            out_ref[...] = jnp.transpose(x4).astype(jnp.bfloat16)


def _relayout_tables(tables):
    t2 = jnp.transpose(tables, (0, 2, 1))
    lin = pl.pallas_call(
        _trans_body,
        grid=(NUM_CAT, NS_TR),
        in_specs=[pl.BlockSpec((1, EMB_DIM, VOCAB), lambda f, s: (f, 0, 0))],
        out_specs=pl.BlockSpec((SLAB, 128), lambda f, s: (f * NS_TR + s, 0)),
        out_shape=jax.ShapeDtypeStruct((NUM_CAT * QA, 128), jnp.bfloat16),
    )(t2)
    return lin.reshape(NUM_CAT * QA * 4, EMB_DIM)


def _mm_body(cont_ref, cat_ref, wc_ref, bc_ref, w0_ref, w2_ref, bk_ref,
             out_ref):
    ce = jnp.dot(cont_ref[...], wc_ref[...],
                 preferred_element_type=jnp.float32) + bc_ref[...]
    acc = jnp.dot(ce, w0_ref[...], preferred_element_type=jnp.float32)
    acc += jnp.dot(cat_ref[...], w2_ref[...],
                   preferred_element_type=jnp.float32)
    out_ref[...] = acc + bk_ref[...]


BM = 1024


def kernel(continuous_static, categorical_static, tables, W_cont, b_cont,
           W_comb, b_comb):
    offs = (jnp.arange(NUM_CAT, dtype=jnp.int32) * (4 * QA))[None, :]
    v = categorical_static.astype(jnp.int32)
    a = jnp.minimum(v // QA, 3)
    rows = offs + 4 * (v - a * QA) + a
    idx2d = rows.reshape(ROWS // 128, 128)
    table_flat = _relayout_tables(tables)
    cat_flat = _gather(table_flat, idx2d).reshape(B, NUM_CAT * EMB_DIM)

    w0 = W_comb[:CONT_DIM]
    w2 = W_comb[CONT_DIM:].astype(jnp.bfloat16)
    return pl.pallas_call(
        _mm_body,
        grid=(B // BM,),
        in_specs=[
            pl.BlockSpec((BM, NUM_CONT), lambda i: (i, 0)),
            pl.BlockSpec((BM, NUM_CAT * EMB_DIM), lambda i: (i, 0)),
            pl.BlockSpec((NUM_CONT, CONT_DIM), lambda i: (0, 0)),
            pl.BlockSpec((1, CONT_DIM), lambda i: (0, 0)),
            pl.BlockSpec((CONT_DIM, HIDDEN), lambda i: (0, 0)),
            pl.BlockSpec((TOTAL - CONT_DIM, HIDDEN), lambda i: (0, 0)),
            pl.BlockSpec((1, HIDDEN), lambda i: (0, 0)),
        ],
        out_specs=pl.BlockSpec((BM, HIDDEN), lambda i: (i, 0)),
        out_shape=jax.ShapeDtypeStruct((B, HIDDEN), jnp.float32),
    )(continuous_static, cat_flat, W_cont, b_cont.reshape(1, CONT_DIM),
      w0, w2, b_comb.reshape(1, HIDDEN))


# speedup vs baseline: 1.0887x; 1.0887x over previous
import functools

import jax
import jax.numpy as jnp
from jax import lax
from jax.experimental import pallas as pl
from jax.experimental.pallas import tpu as pltpu
from jax.experimental.pallas import tpu_sc as plsc

B = 16384
NUM_CONT = 16
NUM_CAT = 26
VOCAB = 100000
HIDDEN = 128
EMB_DIM = HIDDEN // 4
CONT_DIM = HIDDEN // 2
TOTAL = CONT_DIM + EMB_DIM * NUM_CAT

NC = 2
NS = 16
NW = NC * NS
RECS = (B // 8) * 224
R_PER_W = RECS // NW
CHUNK = 1024
N_CHUNKS = R_PER_W // CHUNK
IDX_ROWS = CHUNK // 128


def _gather_body(table_hbm, idx_hbm, out_hbm, idx_v, rows_v, sem):
    wid = lax.axis_index("s") * NC + lax.axis_index("c")
    base = wid * R_PER_W
    ibase = wid * (R_PER_W // 128)

    def body(c, carry):
        row0 = base + c * CHUNK
        irow0 = ibase + c * IDX_ROWS
        pltpu.sync_copy(idx_hbm.at[pl.ds(irow0, IDX_ROWS)], idx_v)
        copies = [
            pltpu.async_copy(
                table_hbm.at[idx_v.at[j]],
                rows_v.at[pl.ds(j * 128, 128)],
                sem,
            )
            for j in range(IDX_ROWS)
        ]
        for cp in copies:
            cp.wait()
        pltpu.sync_copy(rows_v, out_hbm.at[pl.ds(row0, CHUNK)])
        return carry

    lax.fori_loop(0, N_CHUNKS, body, 0)


_gather = functools.partial(
    pl.kernel,
    out_type=jax.ShapeDtypeStruct((RECS, EMB_DIM), jnp.float32),
    mesh=plsc.VectorSubcoreMesh(core_axis_name="c", subcore_axis_name="s"),
    scratch_types=[
        pltpu.VMEM((IDX_ROWS, 128), jnp.int32),
        pltpu.VMEM((CHUNK, EMB_DIM), jnp.float32),
        pltpu.SemaphoreType.DMA,
    ],
    compiler_params=pltpu.CompilerParams(use_tc_tiling_on_sc=False),
)(_gather_body)


QA = 25088
NS_TR = 7
SLAB = QA // NS_TR


def _trans_body(in_ref, out_ref):
    s = pl.program_id(1)
    for k in range(NS_TR):
        @pl.when(s == k)
        def _(k=k):
            parts = []
            for a in range(4):
                lo = a * QA + k * SLAB
                hi = lo + SLAB
                if hi <= VOCAB:
                    parts.append(in_ref[0, :, lo:hi])
                else:
                    tail = in_ref[0, :, lo:VOCAB]
                    parts.append(jnp.concatenate(
                        [tail, jnp.zeros((EMB_DIM, hi - VOCAB),
                                         jnp.float32)], axis=1))
            x4 = jnp.concatenate(parts, axis=0)
            out_ref[...] = jnp.transpose(x4)


def _relayout_tables(tables):
    t2 = jnp.transpose(tables, (0, 2, 1))
    lin = pl.pallas_call(
        _trans_body,
        grid=(NUM_CAT, NS_TR),
        in_specs=[pl.BlockSpec((1, EMB_DIM, VOCAB), lambda f, s: (f, 0, 0))],
        out_specs=pl.BlockSpec((SLAB, 128), lambda f, s: (f * NS_TR + s, 0)),
        out_shape=jax.ShapeDtypeStruct((NUM_CAT * QA, 128), jnp.float32),
    )(t2)
    return lin.reshape(NUM_CAT * QA * 4, EMB_DIM)


def _mm_body(cont_ref, cat_ref, wc_ref, bc_ref, w0_ref, w2_ref, bk_ref,
             out_ref):
    ce = jnp.dot(cont_ref[...], wc_ref[...],
                 preferred_element_type=jnp.float32) + bc_ref[...]
    acc = jnp.dot(ce, w0_ref[...], preferred_element_type=jnp.float32)
    acc += jnp.dot(cat_ref[...], w2_ref[...],
                   preferred_element_type=jnp.float32)
    out_ref[...] = acc + bk_ref[...]


BM = 1024


def kernel(continuous_static, categorical_static, tables, W_cont, b_cont,
           W_comb, b_comb):
    offs = (jnp.arange(NUM_CAT, dtype=jnp.int32) * (4 * QA))[None, :]
    v = categorical_static.astype(jnp.int32)
    a = jnp.minimum(v // QA, 3)
    cat_rows = offs + 4 * (v - a * QA) + a
    padded = jnp.concatenate(
        [jnp.zeros((B, 2), jnp.int32), cat_rows], axis=1)
    idxfull = padded.reshape(B // 8, 8, 7, 4).transpose(0, 2, 1, 3)
    idx2d = idxfull.reshape(RECS // 128, 128)
    table_flat = _relayout_tables(tables)
    g = _gather(table_flat, idx2d)
    xview = (g.reshape(B // 8, 7, 8, 128)
             .transpose(0, 2, 1, 3).reshape(B, TOTAL))
    w0 = W_comb[:CONT_DIM]
    w2z = jnp.concatenate(
        [jnp.zeros((CONT_DIM, HIDDEN), jnp.float32), W_comb[CONT_DIM:]],
        axis=0)

    return pl.pallas_call(
        _mm_body,
        grid=(B // BM,),
        in_specs=[
            pl.BlockSpec((BM, NUM_CONT), lambda i: (i, 0)),
            pl.BlockSpec((BM, TOTAL), lambda i: (i, 0)),
            pl.BlockSpec((NUM_CONT, CONT_DIM), lambda i: (0, 0)),
            pl.BlockSpec((1, CONT_DIM), lambda i: (0, 0)),
            pl.BlockSpec((CONT_DIM, HIDDEN), lambda i: (0, 0)),
            pl.BlockSpec((TOTAL, HIDDEN), lambda i: (0, 0)),
            pl.BlockSpec((1, HIDDEN), lambda i: (0, 0)),
        ],
        out_specs=pl.BlockSpec((BM, HIDDEN), lambda i: (i, 0)),
        out_shape=jax.ShapeDtypeStruct((B, HIDDEN), jnp.float32),
    )(continuous_static, xview, W_cont, b_cont.reshape(1, CONT_DIM),
      w0, w2z, b_comb.reshape(1, HIDDEN))


# speedup vs baseline: 1.8228x; 1.6743x over previous
import functools

import jax
import jax.numpy as jnp
from jax import lax
from jax.experimental import pallas as pl
from jax.experimental.pallas import tpu as pltpu
from jax.experimental.pallas import tpu_sc as plsc

B = 16384
NUM_CONT = 16
NUM_CAT = 26
VOCAB = 100000
HIDDEN = 128
EMB_DIM = HIDDEN // 4
CONT_DIM = HIDDEN // 2
TOTAL = CONT_DIM + EMB_DIM * NUM_CAT

NC = 2
NS = 16
NW = NC * NS
RECS = (B // 8) * 224
R_PER_W = RECS // NW
CHUNK = 1024
N_CHUNKS = R_PER_W // CHUNK
IDX_ROWS = CHUNK // 128


def _gather_body(table_hbm, idx_hbm, out_hbm, idx_v, rows_v, sem):
    wid = lax.axis_index("s") * NC + lax.axis_index("c")
    base = wid * R_PER_W
    ibase = wid * (R_PER_W // 128)

    def body(c, carry):
        row0 = base + c * CHUNK
        irow0 = ibase + c * IDX_ROWS
        pltpu.sync_copy(idx_hbm.at[pl.ds(irow0, IDX_ROWS)], idx_v)
        copies = [
            pltpu.async_copy(
                table_hbm.at[idx_v.at[j]],
                rows_v.at[pl.ds(j * 128, 128)],
                sem,
            )
            for j in range(IDX_ROWS)
        ]
        for cp in copies:
            cp.wait()
        pltpu.sync_copy(rows_v, out_hbm.at[pl.ds(row0, CHUNK)])
        return carry

    lax.fori_loop(0, N_CHUNKS, body, 0)


_gather = functools.partial(
    pl.kernel,
    out_type=jax.ShapeDtypeStruct((RECS, EMB_DIM), jnp.float32),
    mesh=plsc.VectorSubcoreMesh(core_axis_name="c", subcore_axis_name="s"),
    scratch_types=[
        pltpu.VMEM((IDX_ROWS, 128), jnp.int32),
        pltpu.VMEM((CHUNK, EMB_DIM), jnp.float32),
        pltpu.SemaphoreType.DMA,
    ],
    compiler_params=pltpu.CompilerParams(use_tc_tiling_on_sc=False),
)(_gather_body)


QA = 25088
NS_TR = 7
SLAB = QA // NS_TR


def _trans_body(in_ref, out_ref):
    s = pl.program_id(1)
    for k in range(NS_TR):
        @pl.when(s == k)
        def _(k=k):
            parts = []
            for a in range(4):
                lo = a * QA + k * SLAB
                hi = lo + SLAB
                if hi <= VOCAB:
                    parts.append(in_ref[0, :, lo:hi])
                else:
                    tail = in_ref[0, :, lo:VOCAB]
                    parts.append(jnp.concatenate(
                        [tail, jnp.zeros((EMB_DIM, hi - VOCAB),
                                         jnp.float32)], axis=1))
            x4 = jnp.concatenate(parts, axis=0)
            out_ref[...] = jnp.transpose(x4)


def _relayout_tables(tables):
    t2 = jnp.transpose(tables, (0, 2, 1))
    lin = pl.pallas_call(
        _trans_body,
        grid=(NUM_CAT, NS_TR),
        in_specs=[pl.BlockSpec((1, EMB_DIM, VOCAB), lambda f, s: (f, 0, 0))],
        out_specs=pl.BlockSpec((SLAB, 128), lambda f, s: (f * NS_TR + s, 0)),
        out_shape=jax.ShapeDtypeStruct((NUM_CAT * QA, 128), jnp.float32),
    )(t2)
    return lin.reshape(NUM_CAT * QA * 4, EMB_DIM)


def _mm_body(cont_ref, cat_ref, wc_ref, bc_ref, w0_ref, w2_ref, bk_ref,
             out_ref):
    ce = jnp.dot(cont_ref[...], wc_ref[...],
                 preferred_element_type=jnp.float32) + bc_ref[...]
    acc = jnp.dot(ce, w0_ref[...], preferred_element_type=jnp.float32)
    acc += jnp.dot(cat_ref[...], w2_ref[...],
                   preferred_element_type=jnp.float32)
    out_ref[...] = acc + bk_ref[...]


BM = 1024


def kernel(continuous_static, categorical_static, tables, W_cont, b_cont,
           W_comb, b_comb):
    offs = (jnp.arange(NUM_CAT, dtype=jnp.int32) * (4 * QA))[None, :]
    v = categorical_static.astype(jnp.int32)
    a = jnp.minimum(v // QA, 3)
    cat_rows = offs + 4 * (v - a * QA) + a
    padded = jnp.concatenate(
        [cat_rows[:, :2], cat_rows], axis=1)
    idxfull = padded.reshape(B // 8, 8, 7, 4).transpose(0, 2, 1, 3)
    idx2d = idxfull.reshape(RECS // 128, 128)
    table_flat = _relayout_tables(tables)
    g = _gather(table_flat, idx2d)
    xview = (g.reshape(B // 8, 7, 8, 128)
             .transpose(0, 2, 1, 3).reshape(B, TOTAL))
    w0 = W_comb[:CONT_DIM]
    w2z = jnp.concatenate(
        [jnp.zeros((CONT_DIM, HIDDEN), jnp.float32), W_comb[CONT_DIM:]],
        axis=0)

    return pl.pallas_call(
        _mm_body,
        grid=(B // BM,),
        in_specs=[
            pl.BlockSpec((BM, NUM_CONT), lambda i: (i, 0)),
            pl.BlockSpec((BM, TOTAL), lambda i: (i, 0)),
            pl.BlockSpec((NUM_CONT, CONT_DIM), lambda i: (0, 0)),
            pl.BlockSpec((1, CONT_DIM), lambda i: (0, 0)),
            pl.BlockSpec((CONT_DIM, HIDDEN), lambda i: (0, 0)),
            pl.BlockSpec((TOTAL, HIDDEN), lambda i: (0, 0)),
            pl.BlockSpec((1, HIDDEN), lambda i: (0, 0)),
        ],
        out_specs=pl.BlockSpec((BM, HIDDEN), lambda i: (i, 0)),
        out_shape=jax.ShapeDtypeStruct((B, HIDDEN), jnp.float32),
    )(continuous_static, xview, W_cont, b_cont.reshape(1, CONT_DIM),
      w0, w2z, b_comb.reshape(1, HIDDEN))


# speedup vs baseline: 2.1863x; 1.1995x over previous
import functools

import jax
import jax.numpy as jnp
from jax import lax
from jax.experimental import pallas as pl
from jax.experimental.pallas import tpu as pltpu
from jax.experimental.pallas import tpu_sc as plsc

B = 16384
NUM_CONT = 16
NUM_CAT = 26
VOCAB = 100000
HIDDEN = 128
EMB_DIM = HIDDEN // 4
CONT_DIM = HIDDEN // 2
TOTAL = CONT_DIM + EMB_DIM * NUM_CAT

NC = 2
NS = 16
NW = NC * NS
ROWS = B * NUM_CAT
R_PER_W = ROWS // NW
CHUNK = 1024
N_CHUNKS = R_PER_W // CHUNK
IDX_ROWS = CHUNK // 128


def _gather_body(table_hbm, idx_hbm, out_hbm, idx_v, rows_v, sem):
    wid = lax.axis_index("s") * NC + lax.axis_index("c")
    base = wid * R_PER_W
    ibase = wid * (R_PER_W // 128)

    def body(c, carry):
        row0 = base + c * CHUNK
        irow0 = ibase + c * IDX_ROWS
        pltpu.sync_copy(idx_hbm.at[pl.ds(irow0, IDX_ROWS)], idx_v)
        copies = [
            pltpu.async_copy(
                table_hbm.at[idx_v.at[j]],
                rows_v.at[pl.ds(j * 128, 128)],
                sem,
            )
            for j in range(IDX_ROWS)
        ]
        for cp in copies:
            cp.wait()
        pltpu.sync_copy(rows_v, out_hbm.at[pl.ds(row0, CHUNK)])
        return carry

    lax.fori_loop(0, N_CHUNKS, body, 0)


_gather = functools.partial(
    pl.kernel,
    out_type=jax.ShapeDtypeStruct((ROWS, EMB_DIM), jnp.float32),
    mesh=plsc.VectorSubcoreMesh(core_axis_name="c", subcore_axis_name="s"),
    scratch_types=[
        pltpu.VMEM((IDX_ROWS, 128), jnp.int32),
        pltpu.VMEM((CHUNK, EMB_DIM), jnp.float32),
        pltpu.SemaphoreType.DMA,
    ],
    compiler_params=pltpu.CompilerParams(use_tc_tiling_on_sc=False),
)(_gather_body)


QA = 25088
NS_TR = 7
SLAB = QA // NS_TR


def _trans_body(in_ref, out_ref):
    s = pl.program_id(1)
    for k in range(NS_TR):
        @pl.when(s == k)
        def _(k=k):
            parts = []
            for a in range(4):
                lo = a * QA + k * SLAB
                hi = lo + SLAB
                if hi <= VOCAB:
                    parts.append(in_ref[0, :, lo:hi])
                else:
                    tail = in_ref[0, :, lo:VOCAB]
                    parts.append(jnp.concatenate(
                        [tail, jnp.zeros((EMB_DIM, hi - VOCAB),
                                         jnp.float32)], axis=1))
            x4 = jnp.concatenate(parts, axis=0)
            out_ref[...] = jnp.transpose(x4)


def _relayout_tables(tables):
    t2 = jnp.transpose(tables, (0, 2, 1))
    lin = pl.pallas_call(
        _trans_body,
        grid=(NUM_CAT, NS_TR),
        in_specs=[pl.BlockSpec((1, EMB_DIM, VOCAB), lambda f, s: (f, 0, 0))],
        out_specs=pl.BlockSpec((SLAB, 128), lambda f, s: (f * NS_TR + s, 0)),
        out_shape=jax.ShapeDtypeStruct((NUM_CAT * QA, 128), jnp.float32),
    )(t2)
    return lin.reshape(NUM_CAT * QA * 4, EMB_DIM)


def _mm_body(cont_ref, cat_ref, wc_ref, bc_ref, wk_ref, bk_ref, out_ref):
    ce = jnp.dot(cont_ref[...], wc_ref[...],
                 preferred_element_type=jnp.float32) + bc_ref[...]
    x = jnp.concatenate([ce, cat_ref[...]], axis=1)
    out_ref[...] = jnp.dot(x, wk_ref[...],
                           preferred_element_type=jnp.float32) + bk_ref[...]


BM = 1024


def kernel(continuous_static, categorical_static, tables, W_cont, b_cont,
           W_comb, b_comb):
    offs = (jnp.arange(NUM_CAT, dtype=jnp.int32) * (4 * QA))[None, :]
    v = categorical_static.astype(jnp.int32)
    a = jnp.minimum(v // QA, 3)
    rows = offs + 4 * (v - a * QA) + a
    idx2d = rows.reshape(ROWS // 128, 128)
    table_flat = _relayout_tables(tables)
    cat_flat = _gather(table_flat, idx2d).reshape(B, NUM_CAT * EMB_DIM)

    return pl.pallas_call(
        _mm_body,
        grid=(B // BM,),
        in_specs=[
            pl.BlockSpec((BM, NUM_CONT), lambda i: (i, 0)),
            pl.BlockSpec((BM, NUM_CAT * EMB_DIM), lambda i: (i, 0)),
            pl.BlockSpec((NUM_CONT, CONT_DIM), lambda i: (0, 0)),
            pl.BlockSpec((1, CONT_DIM), lambda i: (0, 0)),
            pl.BlockSpec((TOTAL, HIDDEN), lambda i: (0, 0)),
            pl.BlockSpec((1, HIDDEN), lambda i: (0, 0)),
        ],
        out_specs=pl.BlockSpec((BM, HIDDEN), lambda i: (i, 0)),
        out_shape=jax.ShapeDtypeStruct((B, HIDDEN), jnp.float32),
    )(continuous_static, cat_flat, W_cont, b_cont.reshape(1, CONT_DIM),
      W_comb, b_comb.reshape(1, HIDDEN))
